# baseline (speedup 1.0000x reference)
import functools

import jax
import jax.numpy as jnp
from jax import lax
from jax.experimental import pallas as pl
from jax.experimental.pallas import tpu as pltpu
from jax.experimental.pallas import tpu_sc as plsc

LANES = 16
CHUNK = 128
NBUF = 4
NW = 32


def kernel(input_BC, table):
    B, C = input_BC.shape
    V = table.shape[1]
    N = B * C
    per_w = N // NW
    n_chunks = per_w // CHUNK

    idx_flat = input_BC.reshape(N).astype(jnp.int32)
    table128 = table.reshape(table.shape[0] // 2, 2 * V)

    mesh = plsc.VectorSubcoreMesh(core_axis_name="c", subcore_axis_name="s")
    cp = pltpu.CompilerParams(
        needs_layout_passes=False, use_tc_tiling_on_sc=True
    )

    scratch = (
        [pltpu.VMEM((CHUNK,), jnp.int32) for _ in range(NBUF)]
        + [pltpu.VMEM((CHUNK,), jnp.int32) for _ in range(NBUF)]
        + [pltpu.VMEM((CHUNK, 2 * V), jnp.float32) for _ in range(NBUF)]
        + [pltpu.VMEM((CHUNK // 2, 2 * V), jnp.float32) for _ in range(NBUF)]
        + [pltpu.SemaphoreType.DMA for _ in range(3 * NBUF)]
    )

    @functools.partial(
        pl.kernel,
        out_type=jax.ShapeDtypeStruct((N // 2, 2 * V), jnp.float32),
        mesh=mesh,
        compiler_params=cp,
        scratch_types=scratch,
    )
    def embed(table_hbm, idx_hbm, out_hbm, *rest):
        ibuf = rest[:NBUF]
        mbuf = rest[NBUF:2 * NBUF]
        gbuf = rest[2 * NBUF:3 * NBUF]
        obuf = rest[3 * NBUF:4 * NBUF]
        sems = rest[4 * NBUF:]
        isem = sems[:NBUF]
        gsem = sems[NBUF:2 * NBUF]
        ssem = sems[2 * NBUF:3 * NBUF]

        wid = lax.axis_index("s") * 2 + lax.axis_index("c")
        base = wid * per_w
        obase = wid * (per_w // 2)

        def stage_desc(g, b):
            return pltpu.make_async_copy(
                idx_hbm.at[pl.ds(base + g * CHUNK, CHUNK)], ibuf[b], isem[b]
            )

        def gather_desc(b):
            return pltpu.make_async_copy(
                table_hbm.at[mbuf[b]], gbuf[b], gsem[b]
            )

        def store_desc(g, b):
            return pltpu.make_async_copy(
                obuf[b], out_hbm.at[pl.ds(obase + g * (CHUNK // 2), CHUNK // 2)],
                ssem[b],
            )

        def compute_midx(b):
            for t in range(CHUNK // LANES):
                sl = pl.ds(t * LANES, LANES)
                mbuf[b][sl] = lax.shift_right_logical(ibuf[b][sl], 1)

        lane = lax.iota(jnp.int32, LANES)
        cmod = jnp.full((LANES,), C, jnp.int32)
        splice_cols = (lane & 1) * V + (V - 1)
        klo = [lane + k * LANES for k in range(4)]

        def compact_chunk(g, b):
            @pl.loop(0, CHUNK // LANES)
            def _(t):
                for j in range(LANES):
                    jj = t * LANES + j
                    jv = jax.lax.broadcast(jj, (LANES,))
                    srcb = (plsc.load_gather(ibuf[b], [jv]) & 1) << 6
                    for k in range(4):
                        v = plsc.load_gather(gbuf[b], [jv, srcb + klo[k]])
                        obuf[b][
                            t * (LANES // 2) + j // 2,
                            pl.ds((j % 2) * V + k * LANES, LANES),
                        ] = v
                rows = (lane + t * LANES) >> 1
                pos = lax.rem(lane + (base + g * CHUNK + t * LANES), cmod)
                plsc.store_scatter(
                    obuf[b], [rows, splice_cols], pos.astype(jnp.float32)
                )

        for c in range(min(3, n_chunks)):
            stage_desc(c, c % NBUF).start()
        for c in range(min(2, n_chunks)):
            stage_desc(c, c % NBUF).wait()
            compute_midx(c % NBUF)
            gather_desc(c % NBUF).start()

        @pl.loop(0, n_chunks, step=NBUF)
        def _(g0):
            for b in range(NBUF):
                g = g0 + b

                gi = g + 3
                @pl.when(gi < n_chunks)
                def _():
                    stage_desc(gi, (b + 3) % NBUF).start()

                gg = g + 2
                bg = (b + 2) % NBUF
                @pl.when(gg < n_chunks)
                def _():
                    stage_desc(gg, bg).wait()
                    compute_midx(bg)
                    gather_desc(bg).start()

                gather_desc(b).wait()
                @pl.when(g >= NBUF)
                def _():
                    store_desc(g - NBUF, b).wait()
                compact_chunk(g, b)
                store_desc(g, b).start()

        for b in range(NBUF):
            store_desc(n_chunks - NBUF + b, b).wait()

    out = embed(table128, idx_flat)
    return out.reshape(B, C, V)


# speedup vs baseline: 1.3369x; 1.3369x over previous
import functools

import jax
import jax.numpy as jnp
from jax import lax
from jax.experimental import pallas as pl
from jax.experimental.pallas import tpu as pltpu
from jax.experimental.pallas import tpu_sc as plsc

LANES = 16
CHUNK = 128
NBUF = 8
LEAD = 4
NW = 32


def kernel(input_BC, table):
    B, C = input_BC.shape
    V = table.shape[1]
    N = B * C
    per_w = N // NW
    n_chunks = per_w // CHUNK

    idx_flat = input_BC.reshape(N).astype(jnp.int32)

    mesh = plsc.VectorSubcoreMesh(core_axis_name="c", subcore_axis_name="s")
    cp = pltpu.CompilerParams(
        needs_layout_passes=False, use_tc_tiling_on_sc=False
    )

    scratch = (
        [pltpu.VMEM((per_w,), jnp.int32)]
        + [pltpu.VMEM((CHUNK, V), jnp.float32) for _ in range(NBUF)]
        + [pltpu.SemaphoreType.DMA for _ in range(2 * NBUF + 1)]
    )

    @functools.partial(
        pl.kernel,
        out_type=jax.ShapeDtypeStruct((N, V), jnp.float32),
        mesh=mesh,
        compiler_params=cp,
        scratch_types=scratch,
    )
    def embed(table_hbm, idx_hbm, out_hbm, idx_v, *rest):
        bufs = rest[:NBUF]
        gsem = rest[NBUF:2 * NBUF]
        ssem = rest[2 * NBUF:3 * NBUF]
        isem = rest[3 * NBUF]

        wid = lax.axis_index("s") * 2 + lax.axis_index("c")
        base = wid * per_w

        pltpu.make_async_copy(
            idx_hbm.at[pl.ds(base, per_w)], idx_v, isem
        ).start()
        pltpu.make_async_copy(
            idx_hbm.at[pl.ds(base, per_w)], idx_v, isem
        ).wait()

        def gather_desc(chunk, b):
            return pltpu.make_async_copy(
                table_hbm.at[idx_v.at[pl.ds(chunk * CHUNK, CHUNK)]],
                bufs[b],
                gsem[b],
            )

        def store_desc(chunk, b):
            return pltpu.make_async_copy(
                bufs[b],
                out_hbm.at[pl.ds(base + chunk * CHUNK, CHUNK)],
                ssem[b],
            )

        for c in range(LEAD):
            gather_desc(c, c % NBUF).start()

        lane = lax.iota(jnp.int32, LANES)
        cols = jnp.full((LANES,), V - 1, jnp.int32)
        cmod = jnp.full((LANES,), C, jnp.int32)

        @pl.loop(0, n_chunks, step=NBUF)
        def _(g0):
            for b in range(NBUF):
                g = g0 + b
                gc = g + LEAD
                bc = (b + LEAD) % NBUF

                @pl.when(jnp.logical_and(gc < n_chunks, gc >= NBUF))
                def _():
                    store_desc(gc - NBUF, bc).wait()
                    gather_desc(gc, bc).start()

                @pl.when(jnp.logical_and(gc < n_chunks, gc < NBUF))
                def _():
                    gather_desc(gc, bc).start()

                gather_desc(g, b).wait()
                row0 = base + g * CHUNK
                for k in range(CHUNK // LANES):
                    rows = lane + (k * LANES)
                    pos = lax.rem(lane + (row0 + k * LANES), cmod)
                    plsc.store_scatter(
                        bufs[b], [rows, cols], pos.astype(jnp.float32)
                    )
                store_desc(g, b).start()

        for b in range(NBUF):
            store_desc(n_chunks - NBUF + b, b).wait()

    out = embed(table, idx_flat)
    return out.reshape(B, C, V)
